# baseline (speedup 1.0000x reference)
import jax
import jax.numpy as jnp
from jax.experimental import pallas as pl
from jax.experimental.pallas import tpu as pltpu

_K = 64
_W = 250
_RW = 2


def _mono(v):
    u = jax.lax.bitcast_convert_type(v, jnp.int32)
    return u ^ (jax.lax.shift_right_arithmetic(u, 31) & jnp.int32(0x7FFFFFFF))


def _rows_kernel(x_ref, o_ref, cand_ref, pos_ref):
    _BIG = jnp.int32(2**30)
    _INT_MIN = jnp.int32(-(2**31))
    x = x_ref[...]
    RW, C, W = x.shape
    mr = 125 if C % 125 == 0 else 128
    R = C // mr

    def cnt(pred):
        return jnp.sum(pred.astype(jnp.int32), axis=(1, 2), keepdims=True)

    x4 = x.reshape(RW, R, mr, W)
    ci = _mono(jnp.max(x4, axis=3))

    zero3 = jnp.zeros((RW, 1, 1), jnp.int32)
    tc = jnp.where(cnt(ci >= 0) >= _K, zero3, zero3 + _INT_MIN)
    for b in range(30, -1, -1):
        t_try = tc + jnp.int32(2**b)
        tc = jnp.where(cnt(ci >= t_try) >= _K, t_try, tc)
    s_sel = cnt(ci >= tc)

    _OFF = jnp.int32(8192)
    chunk_iota = (jax.lax.broadcasted_iota(jnp.int32, (RW, R, mr), 1) * mr
                  + jax.lax.broadcasted_iota(jnp.int32, (RW, R, mr), 2))
    mi = jnp.where(ci > tc, chunk_iota,
                   jnp.where(ci == tc, chunk_iota + _OFF, _BIG))
    for k in range(_K):
        pv = jnp.min(mi, axis=(1, 2), keepdims=True)
        p0 = pv[0, 0, 0] & jnp.int32(8191)
        p1 = pv[1, 0, 0] & jnp.int32(8191)
        pos_ref[0, k] = p0
        pos_ref[1, k] = p1
        cand_ref[0, k, :] = x_ref[0, pl.ds(p0, 1), :].reshape(W)
        cand_ref[1, k, :] = x_ref[1, pl.ds(p1, 1), :].reshape(W)
        mi = jnp.where(mi == pv, _BIG, mi)

    candi = _mono(cand_ref[...])
    tt = jnp.where(cnt(candi >= 0) >= _K, zero3, zero3 + _INT_MIN)
    for b in range(30, -1, -1):
        t_try = tt + jnp.int32(2**b)
        tt = jnp.where(cnt(candi >= t_try) >= _K, t_try, tt)

    c_above = cnt(candi > tt)
    cnt_eq = cnt(candi == tt)
    e = _K - c_above
    t_f = jax.lax.bitcast_convert_type(
        tt ^ (jax.lax.shift_right_arithmetic(tt, 31) & jnp.int32(0x7FFFFFFF)),
        jnp.float32)

    fast = jnp.logical_and(
        cnt_eq == e,
        jnp.logical_or(tt > tc, s_sel == _K))
    fast_all = jnp.all(fast)

    @pl.when(fast_all)
    def _fast():
        o_ref[...] = (x >= t_f).astype(jnp.float32)

    @pl.when(jnp.logical_not(fast_all))
    def _slow():
        row64 = jax.lax.broadcasted_iota(jnp.int32, (_K, 1), 0)
        cbs = []
        for r in range(RW):
            cb = jnp.zeros((_K, 1), jnp.int32)
            for k in range(_K):
                cb = jnp.where(row64 == k, pos_ref[r, k], cb)
            cbs.append(cb)
        lane = jax.lax.broadcasted_iota(jnp.int32, (RW, _K, W), 2)
        flat = jnp.stack(cbs) * W + lane
        eq = candi == tt

        def i_body(b, lo):
            add = jax.lax.shift_left(jnp.int32(1), jnp.int32(19) - b)
            i_mid = lo + add - 1
            c = cnt(jnp.logical_and(eq, flat <= i_mid))
            return jnp.where(c >= e, lo, lo + add)

        i_e = jax.lax.fori_loop(0, 20, i_body, zero3)
        full_iota = (jax.lax.broadcasted_iota(jnp.int32, (RW, C, W), 1) * W
                     + jax.lax.broadcasted_iota(jnp.int32, (RW, C, W), 2))
        keep = jnp.logical_or(
            x > t_f, jnp.logical_and(x == t_f, full_iota <= i_e))
        o_ref[...] = keep.astype(jnp.float32)


def kernel(logits):
    B, N = logits.shape
    C = N // _W
    x3 = logits.reshape(B, C, _W)
    out = pl.pallas_call(
        _rows_kernel,
        grid=(B // _RW,),
        in_specs=[pl.BlockSpec((_RW, C, _W), lambda i: (i, 0, 0))],
        out_specs=pl.BlockSpec((_RW, C, _W), lambda i: (i, 0, 0)),
        out_shape=jax.ShapeDtypeStruct((B, C, _W), jnp.float32),
        scratch_shapes=[pltpu.VMEM((_RW, _K, _W), jnp.float32),
                        pltpu.SMEM((_RW, _K), jnp.int32)],
        compiler_params=pltpu.CompilerParams(
            dimension_semantics=("arbitrary",),
        ),
    )(x3)
    return out.reshape(B, N)


# speedup vs baseline: 2.9969x; 2.9969x over previous
import jax
import jax.numpy as jnp
from jax.experimental import pallas as pl
from jax.experimental.pallas import tpu as pltpu

_K = 64
_W = 250
_RW = 2


def _mono(v):
    u = jax.lax.bitcast_convert_type(v, jnp.int32)
    return u ^ (jax.lax.shift_right_arithmetic(u, 31) & jnp.int32(0x7FFFFFFF))


def _rows_kernel(x_ref, o_ref, cand_ref, pos_ref):
    _BIG = jnp.int32(2**30)
    _INT_MIN = jnp.int32(-(2**31))
    x = x_ref[...]
    RW, C, W = x.shape
    mr = 125 if C % 125 == 0 else 128
    R = C // mr

    def cnt(pred):
        return jnp.sum(pred.astype(jnp.int32), axis=(1, 2), keepdims=True)

    ci = _mono(jnp.max(x, axis=2)).reshape(RW, R, mr)

    zero3 = jnp.zeros((RW, 1, 1), jnp.int32)
    tc = jnp.where(cnt(ci >= 0) >= _K, zero3, zero3 + _INT_MIN)

    def _quad_search(data, t):
        def q_body(i, t):
            s = jax.lax.shift_left(jnp.int32(1), jnp.int32(29) - 2 * i)
            ge1 = cnt(data >= t + s) >= _K
            ge2 = cnt(data >= t + 2 * s) >= _K
            ge3 = cnt(data >= t + 3 * s) >= _K
            add = jnp.where(ge3, 3 * s,
                            jnp.where(ge2, 2 * s, jnp.where(ge1, s, 0)))
            return t + add

        t = jax.lax.fori_loop(0, 15, q_body, t)
        t1 = t + jnp.int32(1)
        return jnp.where(cnt(data >= t1) >= _K, t1, t)

    tc = _quad_search(ci, tc)
    s_sel = cnt(ci >= tc)

    _OFF = jnp.int32(8192)
    chunk_iota = (jax.lax.broadcasted_iota(jnp.int32, (RW, R, mr), 1) * mr
                  + jax.lax.broadcasted_iota(jnp.int32, (RW, R, mr), 2))
    mi0 = jnp.where(ci > tc, chunk_iota,
                    jnp.where(ci == tc, chunk_iota + _OFF, _BIG))

    def g_body(k2, mi):
        for half in range(2):
            k = 2 * k2 + half
            pv = jnp.min(mi, axis=(1, 2), keepdims=True)
            p0 = pv[0, 0, 0] & jnp.int32(8191)
            p1 = pv[1, 0, 0] & jnp.int32(8191)
            pos_ref[0, k] = p0
            pos_ref[1, k] = p1
            cand_ref[0, pl.ds(k, 1), :] = x_ref[0, pl.ds(p0, 1), :]
            cand_ref[1, pl.ds(k, 1), :] = x_ref[1, pl.ds(p1, 1), :]
            mi = jnp.where(mi == pv, _BIG, mi)
        return mi

    jax.lax.fori_loop(0, _K // 2, g_body, mi0)

    candi = _mono(cand_ref[...])

    tt = jnp.where(cnt(candi >= 0) >= _K, zero3, zero3 + _INT_MIN)
    tt = _quad_search(candi, tt)

    c_above = cnt(candi > tt)
    cnt_eq = cnt(candi == tt)
    e = _K - c_above
    t_f = jax.lax.bitcast_convert_type(
        tt ^ (jax.lax.shift_right_arithmetic(tt, 31) & jnp.int32(0x7FFFFFFF)),
        jnp.float32)

    fast = jnp.logical_and(
        cnt_eq == e,
        jnp.logical_or(tt > tc, s_sel == _K))
    fast_all = jnp.all(fast)

    @pl.when(fast_all)
    def _fast():
        o_ref[...] = (x >= t_f).astype(jnp.float32)

    @pl.when(jnp.logical_not(fast_all))
    def _slow():
        row64 = jax.lax.broadcasted_iota(jnp.int32, (_K, 1), 0)
        cbs = []
        for r in range(RW):
            cb = jnp.zeros((_K, 1), jnp.int32)
            for k in range(_K):
                cb = jnp.where(row64 == k, pos_ref[r, k], cb)
            cbs.append(cb)
        lane = jax.lax.broadcasted_iota(jnp.int32, (RW, _K, W), 2)
        flat = jnp.stack(cbs) * W + lane
        eq = candi == tt

        def i_body(b, lo):
            add = jax.lax.shift_left(jnp.int32(1), jnp.int32(19) - b)
            i_mid = lo + add - 1
            c = cnt(jnp.logical_and(eq, flat <= i_mid))
            return jnp.where(c >= e, lo, lo + add)

        i_e = jax.lax.fori_loop(0, 20, i_body, zero3)
        full_iota = (jax.lax.broadcasted_iota(jnp.int32, (RW, C, W), 1) * W
                     + jax.lax.broadcasted_iota(jnp.int32, (RW, C, W), 2))
        keep = jnp.logical_or(
            x > t_f, jnp.logical_and(x == t_f, full_iota <= i_e))
        o_ref[...] = keep.astype(jnp.float32)


def kernel(logits):
    B, N = logits.shape
    C = N // _W
    x3 = logits.reshape(B, C, _W)
    out = pl.pallas_call(
        _rows_kernel,
        grid=(B // _RW,),
        in_specs=[pl.BlockSpec((_RW, C, _W), lambda i: (i, 0, 0))],
        out_specs=pl.BlockSpec((_RW, C, _W), lambda i: (i, 0, 0)),
        out_shape=jax.ShapeDtypeStruct((B, C, _W), jnp.float32),
        scratch_shapes=[pltpu.VMEM((_RW, _K, _W), jnp.float32),
                        pltpu.SMEM((_RW, _K), jnp.int32)],
        compiler_params=pltpu.CompilerParams(
            dimension_semantics=("arbitrary",),
        ),
    )(x3)
    return out.reshape(B, N)
